# baseline (speedup 1.0000x reference)
import functools

import jax
import jax.numpy as jnp
from jax import lax
from jax.experimental import pallas as pl
from jax.experimental.pallas import tpu as pltpu
from jax.experimental.pallas import tpu_sc as plsc

SCALAR = 20.0
WEIGHT = 0.25
EPS_LS = 0.1

SUB = 1000
UNROLL = 4
KB = SUB * UNROLL


def _make_sc_gather(K, D, B):
    info = plsc.get_sparse_core_info()
    NC, NS, L = info.num_cores, info.num_subcores, info.num_lanes
    NW = NC * NS
    W = 2 * D
    assert W % L == 0 and B % (8 * NW) == 0 and b_ok(B, NW)
    b_per_w = B // NW
    mesh = plsc.VectorSubcoreMesh(core_axis_name="c", subcore_axis_name="s")

    @functools.partial(
        pl.kernel,
        mesh=mesh,
        out_type=[
            jax.ShapeDtypeStruct((B, W), jnp.float32),
            jax.ShapeDtypeStruct((B, W), jnp.float32),
        ],
        scratch_types=[
            pltpu.VMEM((b_per_w,), jnp.int32),
            pltpu.VMEM((b_per_w,), jnp.int32),
            pltpu.VMEM((b_per_w, W), jnp.float32),
            pltpu.VMEM((b_per_w, W), jnp.float32),
            pltpu.SemaphoreType.DMA,
        ],
    )
    def gather2(tab_c_hbm, tab_i_hbm, idx_hbm, out_c, out_i,
                idx_v, half_v, rows_c, rows_i, sem):
        wid = lax.axis_index("s") * NC + lax.axis_index("c")
        base = wid * b_per_w
        pltpu.sync_copy(idx_hbm.at[pl.ds(base, b_per_w)], idx_v)
        for j in range(b_per_w // L):
            sl = pl.ds(j * L, L)
            half_v[sl] = lax.shift_right_logical(idx_v[sl], 1)
        pltpu.async_copy(tab_c_hbm.at[half_v], rows_c, sem).wait()
        pltpu.async_copy(tab_i_hbm.at[half_v], rows_i, sem).wait()
        pltpu.sync_copy(rows_c, out_c.at[pl.ds(base, b_per_w)])
        pltpu.sync_copy(rows_i, out_i.at[pl.ds(base, b_per_w)])

    return gather2


def b_ok(B, NW):
    return (B // NW) % 16 == 0


def _fused_body(x_ref, lc_lo, lc_hi, li_lo, li_hi, out_ref,
                m_c, s_c, m_i, s_i, cs_c, cs_i, hub):
    k = pl.program_id(0)
    nb = pl.num_programs(0)
    K = nb * 2 * lc_lo.shape[0]
    B = x_ref.shape[0]

    @pl.when(k == 0)
    def _init():
        m_c[...] = jnp.full(m_c.shape, -1e30, jnp.float32)
        m_i[...] = jnp.full(m_i.shape, -1e30, jnp.float32)
        s_c[...] = jnp.zeros(s_c.shape, jnp.float32)
        s_i[...] = jnp.zeros(s_i.shape, jnp.float32)
        cs_c[...] = jnp.zeros(cs_c.shape, jnp.float32)
        cs_i[...] = jnp.zeros(cs_i.shape, jnp.float32)
        hub[...] = jnp.zeros(hub.shape, jnp.float32)

    x = x_ref[...]
    dn = (((1,), (1,)), ((), ()))
    xb = x.astype(jnp.bfloat16)
    half = UNROLL // 2
    for u in range(UNROLL):
        lc_r = lc_lo if u < half else lc_hi
        li_r = li_lo if u < half else li_hi
        v = u % half
        lc = lc_r[v * SUB:(v + 1) * SUB, :]
        li = li_r[v * SUB:(v + 1) * SUB, :]
        a = lax.dot_general(xb, lc.astype(jnp.bfloat16), dn,
                            preferred_element_type=jnp.float32) * SCALAR
        c = lax.dot_general(xb, li.astype(jnp.bfloat16), dn,
                            preferred_element_type=jnp.float32) * SCALAR

        m_old = m_c[...]
        m_new = jnp.maximum(m_old, jnp.max(a, axis=1, keepdims=True))
        s_c[...] = (s_c[...] * jnp.exp(m_old - m_new)
                    + jnp.sum(jnp.exp(a - m_new), axis=1, keepdims=True))
        m_c[...] = m_new

        m_old = m_i[...]
        m_new = jnp.maximum(m_old, jnp.max(c, axis=1, keepdims=True))
        s_i[...] = (s_i[...] * jnp.exp(m_old - m_new)
                    + jnp.sum(jnp.exp(c - m_new), axis=1, keepdims=True))
        m_i[...] = m_new

        d = a - c
        ad = jnp.abs(d)
        h = jnp.where(ad < 1.0, 0.5 * d * d, ad - 0.5)
        hub[...] += jnp.sum(h, keepdims=True).reshape(1, 1)

        cs_c[...] += jnp.sum(lc, axis=0, keepdims=True)
        cs_i[...] += jnp.sum(li, axis=0, keepdims=True)

    @pl.when(k == nb - 1)
    def _finalize():
        lse_c = m_c[...] + jnp.log(s_c[...])
        lse_i = m_i[...] + jnp.log(s_i[...])
        sum_c = SCALAR * jnp.sum(x * cs_c[...], axis=1, keepdims=True)
        sum_i = SCALAR * jnp.sum(x * cs_i[...], axis=1, keepdims=True)
        inv_k = jnp.float32(1.0 / K)
        ce_c = jnp.mean(lse_c - EPS_LS * inv_k * sum_c)
        ce_i = jnp.mean(lse_i - EPS_LS * inv_k * sum_i)
        con = jnp.sum(hub[...]) / jnp.float32(B * K)
        out_ref[...] = jnp.broadcast_to(ce_c + ce_i + WEIGHT * con, (1, 1))


def _fused_loss(inputs, lut_ccc, lut_icc):
    B, D = inputs.shape
    K = lut_ccc.shape[0]
    assert K % KB == 0
    nb = K // KB
    out = pl.pallas_call(
        _fused_body,
        grid=(nb,),
        in_specs=[
            pl.BlockSpec((B, D), lambda k: (0, 0)),
            pl.BlockSpec((KB // 2, D), lambda k: (2 * k, 0)),
            pl.BlockSpec((KB // 2, D), lambda k: (2 * k + 1, 0)),
            pl.BlockSpec((KB // 2, D), lambda k: (2 * k, 0)),
            pl.BlockSpec((KB // 2, D), lambda k: (2 * k + 1, 0)),
        ],
        out_specs=pl.BlockSpec((1, 1), lambda k: (0, 0)),
        out_shape=jax.ShapeDtypeStruct((1, 1), jnp.float32),
        scratch_shapes=[
            pltpu.VMEM((B, 1), jnp.float32),
            pltpu.VMEM((B, 1), jnp.float32),
            pltpu.VMEM((B, 1), jnp.float32),
            pltpu.VMEM((B, 1), jnp.float32),
            pltpu.VMEM((1, D), jnp.float32),
            pltpu.VMEM((1, D), jnp.float32),
            pltpu.VMEM((1, 1), jnp.float32),
        ],
    )(inputs, lut_ccc, lut_ccc, lut_icc, lut_icc)
    return out[0, 0]


def _tgt_body(x_ref, gc_ref, gi_ref, t_ref, out_ref):
    x = x_ref[...]
    B, D = x.shape
    odd = (t_ref[...] & 1) == 1
    pc = gc_ref[...]
    pi = gi_ref[...]
    g_c = jnp.where(odd, pc[:, D:], pc[:, :D])
    g_i = jnp.where(odd, pi[:, D:], pi[:, :D])
    tgt_c = jnp.sum(x * g_c, axis=1, keepdims=True)
    tgt_i = jnp.sum(x * g_i, axis=1, keepdims=True)
    coef = jnp.float32(-(1.0 - EPS_LS) * SCALAR)
    out_ref[...] = jnp.broadcast_to(coef * (jnp.mean(tgt_c) + jnp.mean(tgt_i)),
                                    (1, 1))


def _tgt_term(inputs, pairs_c, pairs_i, targets2d):
    out = pl.pallas_call(
        _tgt_body,
        out_shape=jax.ShapeDtypeStruct((1, 1), jnp.float32),
    )(inputs, pairs_c, pairs_i, targets2d)
    return out[0, 0]


def kernel(inputs, targets, lut_ccc, lut_icc):
    B, D = inputs.shape
    K = lut_ccc.shape[0]
    tab_c = lut_ccc.reshape(K // 2, 2 * D)
    tab_i = lut_icc.reshape(K // 2, 2 * D)
    pairs_c, pairs_i = _make_sc_gather(K, D, B)(tab_c, tab_i, targets)
    main = _fused_loss(inputs, lut_ccc, lut_icc)
    tgt = _tgt_term(inputs, pairs_c, pairs_i, targets.reshape(B, 1))
    return main + tgt


# speedup vs baseline: 1.0040x; 1.0040x over previous
import functools

import jax
import jax.numpy as jnp
from jax import lax
from jax.experimental import pallas as pl
from jax.experimental.pallas import tpu as pltpu
from jax.experimental.pallas import tpu_sc as plsc

SCALAR = 20.0
WEIGHT = 0.25
EPS_LS = 0.1

SUB = 1000
UNROLL = 4
KB = SUB * UNROLL


def _make_sc_gather(K, D, B):
    info = plsc.get_sparse_core_info()
    NC, NS, L = info.num_cores, info.num_subcores, info.num_lanes
    NW = NC * NS
    W = 2 * D
    assert W % L == 0 and B % (8 * NW) == 0 and b_ok(B, NW)
    b_per_w = B // NW
    mesh = plsc.VectorSubcoreMesh(core_axis_name="c", subcore_axis_name="s")

    @functools.partial(
        pl.kernel,
        mesh=mesh,
        out_type=[
            jax.ShapeDtypeStruct((B, W), jnp.float32),
            jax.ShapeDtypeStruct((B, W), jnp.float32),
        ],
        scratch_types=[
            pltpu.VMEM((b_per_w,), jnp.int32),
            pltpu.VMEM((b_per_w,), jnp.int32),
            pltpu.VMEM((b_per_w, W), jnp.float32),
            pltpu.VMEM((b_per_w, W), jnp.float32),
            pltpu.SemaphoreType.DMA,
        ],
    )
    def gather2(tab_c_hbm, tab_i_hbm, idx_hbm, out_c, out_i,
                idx_v, half_v, rows_c, rows_i, sem):
        wid = lax.axis_index("s") * NC + lax.axis_index("c")
        base = wid * b_per_w
        pltpu.sync_copy(idx_hbm.at[pl.ds(base, b_per_w)], idx_v)
        for j in range(b_per_w // L):
            sl = pl.ds(j * L, L)
            half_v[sl] = lax.shift_right_logical(idx_v[sl], 1)
        pltpu.async_copy(tab_c_hbm.at[half_v], rows_c, sem).wait()
        pltpu.async_copy(tab_i_hbm.at[half_v], rows_i, sem).wait()
        pltpu.sync_copy(rows_c, out_c.at[pl.ds(base, b_per_w)])
        pltpu.sync_copy(rows_i, out_i.at[pl.ds(base, b_per_w)])

    return gather2


def b_ok(B, NW):
    return (B // NW) % 16 == 0


def _fused_body(x_ref, lc_ref, li_ref, gc_ref, gi_ref, t_ref, out_ref,
                m_c, s_c, m_i, s_i, cs_c, cs_i, hub):
    k = pl.program_id(0)
    nb = pl.num_programs(0)
    K = nb * lc_ref.shape[0]
    B = x_ref.shape[0]

    @pl.when(k == 0)
    def _init():
        m_c[...] = jnp.full(m_c.shape, -1e30, jnp.float32)
        m_i[...] = jnp.full(m_i.shape, -1e30, jnp.float32)
        s_c[...] = jnp.zeros(s_c.shape, jnp.float32)
        s_i[...] = jnp.zeros(s_i.shape, jnp.float32)
        cs_c[...] = jnp.zeros(cs_c.shape, jnp.float32)
        cs_i[...] = jnp.zeros(cs_i.shape, jnp.float32)
        hub[...] = jnp.zeros(hub.shape, jnp.float32)

    x = x_ref[...]
    dn = (((1,), (1,)), ((), ()))
    xb = x.astype(jnp.bfloat16)
    for u in range(UNROLL):
        lc = lc_ref[u * SUB:(u + 1) * SUB, :]
        li = li_ref[u * SUB:(u + 1) * SUB, :]
        a = lax.dot_general(xb, lc.astype(jnp.bfloat16), dn,
                            preferred_element_type=jnp.float32) * SCALAR
        c = lax.dot_general(xb, li.astype(jnp.bfloat16), dn,
                            preferred_element_type=jnp.float32) * SCALAR

        m_old = m_c[...]
        m_new = jnp.maximum(m_old, jnp.max(a, axis=1, keepdims=True))
        s_c[...] = (s_c[...] * jnp.exp(m_old - m_new)
                    + jnp.sum(jnp.exp(a - m_new), axis=1, keepdims=True))
        m_c[...] = m_new

        m_old = m_i[...]
        m_new = jnp.maximum(m_old, jnp.max(c, axis=1, keepdims=True))
        s_i[...] = (s_i[...] * jnp.exp(m_old - m_new)
                    + jnp.sum(jnp.exp(c - m_new), axis=1, keepdims=True))
        m_i[...] = m_new

        d = a - c
        ad = jnp.abs(d)
        h = jnp.where(ad < 1.0, 0.5 * d * d, ad - 0.5)
        hub[...] += jnp.sum(h, keepdims=True).reshape(1, 1)

        cs_c[...] += jnp.sum(lc, axis=0, keepdims=True)
        cs_i[...] += jnp.sum(li, axis=0, keepdims=True)

    @pl.when(k == nb - 1)
    def _finalize():
        lse_c = m_c[...] + jnp.log(s_c[...])
        lse_i = m_i[...] + jnp.log(s_i[...])
        D = x.shape[1]
        odd = (t_ref[...] & 1) == 1
        pc = gc_ref[...]
        pi = gi_ref[...]
        g_c = jnp.where(odd, pc[:, D:], pc[:, :D])
        g_i = jnp.where(odd, pi[:, D:], pi[:, :D])
        tgt_c = SCALAR * jnp.sum(x * g_c, axis=1, keepdims=True)
        tgt_i = SCALAR * jnp.sum(x * g_i, axis=1, keepdims=True)
        sum_c = SCALAR * jnp.sum(x * cs_c[...], axis=1, keepdims=True)
        sum_i = SCALAR * jnp.sum(x * cs_i[...], axis=1, keepdims=True)
        inv_k = jnp.float32(1.0 / K)
        ce_c = jnp.mean(lse_c - (1.0 - EPS_LS) * tgt_c - EPS_LS * inv_k * sum_c)
        ce_i = jnp.mean(lse_i - (1.0 - EPS_LS) * tgt_i - EPS_LS * inv_k * sum_i)
        con = jnp.sum(hub[...]) / jnp.float32(B * K)
        out_ref[...] = jnp.broadcast_to(ce_c + ce_i + WEIGHT * con, (1, 1))


def _fused_loss(inputs, lut_ccc, lut_icc, pairs_c, pairs_i, targets2d):
    B, D = inputs.shape
    K = lut_ccc.shape[0]
    assert K % KB == 0
    nb = K // KB
    out = pl.pallas_call(
        _fused_body,
        grid=(nb,),
        in_specs=[
            pl.BlockSpec((B, D), lambda k: (0, 0)),
            pl.BlockSpec((KB, D), lambda k: (k, 0)),
            pl.BlockSpec((KB, D), lambda k: (k, 0)),
            pl.BlockSpec((B, 2 * D), lambda k: (0, 0)),
            pl.BlockSpec((B, 2 * D), lambda k: (0, 0)),
            pl.BlockSpec((B, 1), lambda k: (0, 0)),
        ],
        out_specs=pl.BlockSpec((1, 1), lambda k: (0, 0)),
        out_shape=jax.ShapeDtypeStruct((1, 1), jnp.float32),
        scratch_shapes=[
            pltpu.VMEM((B, 1), jnp.float32),
            pltpu.VMEM((B, 1), jnp.float32),
            pltpu.VMEM((B, 1), jnp.float32),
            pltpu.VMEM((B, 1), jnp.float32),
            pltpu.VMEM((1, D), jnp.float32),
            pltpu.VMEM((1, D), jnp.float32),
            pltpu.VMEM((1, 1), jnp.float32),
        ],
    )(inputs, lut_ccc, lut_icc, pairs_c, pairs_i, targets2d)
    return out[0, 0]


def kernel(inputs, targets, lut_ccc, lut_icc):
    B, D = inputs.shape
    K = lut_ccc.shape[0]
    tab_c = lut_ccc.reshape(K // 2, 2 * D)
    tab_i = lut_icc.reshape(K // 2, 2 * D)
    pairs_c, pairs_i = _make_sc_gather(K, D, B)(tab_c, tab_i, targets)
    return _fused_loss(inputs, lut_ccc, lut_icc, pairs_c, pairs_i,
                       targets.reshape(B, 1))


# speedup vs baseline: 1.0529x; 1.0487x over previous
import functools

import jax
import jax.numpy as jnp
from jax import lax
from jax.experimental import pallas as pl
from jax.experimental.pallas import tpu as pltpu
from jax.experimental.pallas import tpu_sc as plsc

SCALAR = 20.0
WEIGHT = 0.25
EPS_LS = 0.1

SUB = 500
UNROLL = 8
KB = SUB * UNROLL


def _make_sc_gather(K, D, B):
    info = plsc.get_sparse_core_info()
    NC, NS, L = info.num_cores, info.num_subcores, info.num_lanes
    NW = NC * NS
    W = 2 * D
    assert W % L == 0 and B % (8 * NW) == 0 and b_ok(B, NW)
    b_per_w = B // NW
    mesh = plsc.VectorSubcoreMesh(core_axis_name="c", subcore_axis_name="s")

    @functools.partial(
        pl.kernel,
        mesh=mesh,
        out_type=[
            jax.ShapeDtypeStruct((B, W), jnp.float32),
            jax.ShapeDtypeStruct((B, W), jnp.float32),
        ],
        scratch_types=[
            pltpu.VMEM((b_per_w,), jnp.int32),
            pltpu.VMEM((b_per_w,), jnp.int32),
            pltpu.VMEM((b_per_w, W), jnp.float32),
            pltpu.VMEM((b_per_w, W), jnp.float32),
            pltpu.SemaphoreType.DMA,
        ],
    )
    def gather2(tab_c_hbm, tab_i_hbm, idx_hbm, out_c, out_i,
                idx_v, half_v, rows_c, rows_i, sem):
        wid = lax.axis_index("s") * NC + lax.axis_index("c")
        base = wid * b_per_w
        pltpu.sync_copy(idx_hbm.at[pl.ds(base, b_per_w)], idx_v)
        for j in range(b_per_w // L):
            sl = pl.ds(j * L, L)
            half_v[sl] = lax.shift_right_logical(idx_v[sl], 1)
        pltpu.async_copy(tab_c_hbm.at[half_v], rows_c, sem).wait()
        pltpu.async_copy(tab_i_hbm.at[half_v], rows_i, sem).wait()
        pltpu.sync_copy(rows_c, out_c.at[pl.ds(base, b_per_w)])
        pltpu.sync_copy(rows_i, out_i.at[pl.ds(base, b_per_w)])

    return gather2


def b_ok(B, NW):
    return (B // NW) % 16 == 0


def _fused_body(x_ref, lc_ref, li_ref, gc_ref, gi_ref, t_ref, out_ref,
                m_c, s_c, m_i, s_i, cs_c, cs_i, hub):
    k = pl.program_id(0)
    nb = pl.num_programs(0)
    K = nb * lc_ref.shape[0]
    B = x_ref.shape[0]

    @pl.when(k == 0)
    def _init():
        m_c[...] = jnp.full(m_c.shape, -1e30, jnp.float32)
        m_i[...] = jnp.full(m_i.shape, -1e30, jnp.float32)
        s_c[...] = jnp.zeros(s_c.shape, jnp.float32)
        s_i[...] = jnp.zeros(s_i.shape, jnp.float32)
        cs_c[...] = jnp.zeros(cs_c.shape, jnp.float32)
        cs_i[...] = jnp.zeros(cs_i.shape, jnp.float32)
        hub[...] = jnp.zeros(hub.shape, jnp.float32)

    x = x_ref[...]
    dn = (((1,), (1,)), ((), ()))
    xb = x.astype(jnp.bfloat16)
    for u in range(UNROLL):
        lc = lc_ref[u * SUB:(u + 1) * SUB, :]
        li = li_ref[u * SUB:(u + 1) * SUB, :]
        a = lax.dot_general(xb, lc.astype(jnp.bfloat16), dn,
                            preferred_element_type=jnp.float32) * SCALAR
        c = lax.dot_general(xb, li.astype(jnp.bfloat16), dn,
                            preferred_element_type=jnp.float32) * SCALAR

        m_old = m_c[...]
        m_new = jnp.maximum(m_old, jnp.max(a, axis=1, keepdims=True))
        s_c[...] = (s_c[...] * jnp.exp(m_old - m_new)
                    + jnp.sum(jnp.exp(a - m_new), axis=1, keepdims=True))
        m_c[...] = m_new

        m_old = m_i[...]
        m_new = jnp.maximum(m_old, jnp.max(c, axis=1, keepdims=True))
        s_i[...] = (s_i[...] * jnp.exp(m_old - m_new)
                    + jnp.sum(jnp.exp(c - m_new), axis=1, keepdims=True))
        m_i[...] = m_new

        d = a - c
        ad = jnp.abs(d)
        h = jnp.where(ad < 1.0, 0.5 * d * d, ad - 0.5)
        hub[...] += jnp.sum(h, keepdims=True).reshape(1, 1)

        cs_c[...] += jnp.sum(lc, axis=0, keepdims=True)
        cs_i[...] += jnp.sum(li, axis=0, keepdims=True)

    @pl.when(k == nb - 1)
    def _finalize():
        lse_c = m_c[...] + jnp.log(s_c[...])
        lse_i = m_i[...] + jnp.log(s_i[...])
        D = x.shape[1]
        odd = (t_ref[...] & 1) == 1
        pc = gc_ref[...]
        pi = gi_ref[...]
        g_c = jnp.where(odd, pc[:, D:], pc[:, :D])
        g_i = jnp.where(odd, pi[:, D:], pi[:, :D])
        tgt_c = SCALAR * jnp.sum(x * g_c, axis=1, keepdims=True)
        tgt_i = SCALAR * jnp.sum(x * g_i, axis=1, keepdims=True)
        sum_c = SCALAR * jnp.sum(x * cs_c[...], axis=1, keepdims=True)
        sum_i = SCALAR * jnp.sum(x * cs_i[...], axis=1, keepdims=True)
        inv_k = jnp.float32(1.0 / K)
        ce_c = jnp.mean(lse_c - (1.0 - EPS_LS) * tgt_c - EPS_LS * inv_k * sum_c)
        ce_i = jnp.mean(lse_i - (1.0 - EPS_LS) * tgt_i - EPS_LS * inv_k * sum_i)
        con = jnp.sum(hub[...]) / jnp.float32(B * K)
        out_ref[...] = jnp.broadcast_to(ce_c + ce_i + WEIGHT * con, (1, 1))


def _fused_loss(inputs, lut_ccc, lut_icc, pairs_c, pairs_i, targets2d):
    B, D = inputs.shape
    K = lut_ccc.shape[0]
    assert K % KB == 0
    nb = K // KB
    out = pl.pallas_call(
        _fused_body,
        grid=(nb,),
        in_specs=[
            pl.BlockSpec((B, D), lambda k: (0, 0)),
            pl.BlockSpec((KB, D), lambda k: (k, 0)),
            pl.BlockSpec((KB, D), lambda k: (k, 0)),
            pl.BlockSpec((B, 2 * D), lambda k: (0, 0)),
            pl.BlockSpec((B, 2 * D), lambda k: (0, 0)),
            pl.BlockSpec((B, 1), lambda k: (0, 0)),
        ],
        out_specs=pl.BlockSpec((1, 1), lambda k: (0, 0)),
        out_shape=jax.ShapeDtypeStruct((1, 1), jnp.float32),
        scratch_shapes=[
            pltpu.VMEM((B, 1), jnp.float32),
            pltpu.VMEM((B, 1), jnp.float32),
            pltpu.VMEM((B, 1), jnp.float32),
            pltpu.VMEM((B, 1), jnp.float32),
            pltpu.VMEM((1, D), jnp.float32),
            pltpu.VMEM((1, D), jnp.float32),
            pltpu.VMEM((1, 1), jnp.float32),
        ],
    )(inputs, lut_ccc, lut_icc, pairs_c, pairs_i, targets2d)
    return out[0, 0]


def kernel(inputs, targets, lut_ccc, lut_icc):
    B, D = inputs.shape
    K = lut_ccc.shape[0]
    tab_c = lut_ccc.reshape(K // 2, 2 * D)
    tab_i = lut_icc.reshape(K // 2, 2 * D)
    pairs_c, pairs_i = _make_sc_gather(K, D, B)(tab_c, tab_i, targets)
    return _fused_loss(inputs, lut_ccc, lut_icc, pairs_c, pairs_i,
                       targets.reshape(B, 1))
